# baseline (speedup 1.0000x reference)
import functools

import jax
import jax.numpy as jnp
from jax import lax
from jax.experimental import pallas as pl
from jax.experimental.pallas import tpu as pltpu
from jax.experimental.pallas import tpu_sc as plsc

_ROW = 128
_K = 8
_NBUF = 2
_NT = 2
_RBL = 2048


@functools.lru_cache(maxsize=None)
def _make_table_transpose(V, D):
    nq = 128 // D
    nb = -(-V // (nq * _RBL))

    def body(x_ref, o_ref):
        x = x_ref[...]
        o_ref[...] = jnp.concatenate(
            [x[:, j * _RBL:(j + 1) * _RBL] for j in range(nq)], axis=0
        ).T

    return pl.pallas_call(
        body,
        grid=(nb,),
        in_specs=[pl.BlockSpec((D, nq * _RBL), lambda b: (0, b))],
        out_specs=pl.BlockSpec((_RBL, 128), lambda b: (b, 0)),
        out_shape=jax.ShapeDtypeStruct((nb * _RBL, 128), jnp.float32),
    )


@functools.lru_cache(maxsize=None)
def _make_gather(V, D, B, H):
    info = plsc.get_sparse_core_info()
    nw = info.num_cores * info.num_subcores
    nbatch = B // H
    ngrp_b = nbatch // _ROW
    rows_per_w = B // (nw * _ROW)
    n_chunks = rows_per_w // _K
    assert rows_per_w % _K == 0 and nbatch % _ROW == 0 and n_chunks % 2 == 1
    chunk = _K * _ROW
    ncg = D // 8
    mesh = plsc.VectorSubcoreMesh(core_axis_name="c", subcore_axis_name="s")

    @functools.partial(
        pl.kernel,
        mesh=mesh,
        compiler_params=pltpu.CompilerParams(
            use_tc_tiling_on_sc=False, needs_layout_passes=False
        ),
        out_type=jax.ShapeDtypeStruct((B, D), jnp.float32),
        scratch_types=[
            pltpu.VMEM((rows_per_w, _ROW), jnp.int32),
            pltpu.VMEM((_NBUF, chunk, D), jnp.float32),
            pltpu.VMEM((_NT, ncg, D, D), jnp.float32),
            pltpu.SemaphoreType.DMA((_NBUF,)),
            pltpu.SemaphoreType.DMA((_NT,)),
        ],
    )
    def k(idx_hbm, table_hbm, out_hbm, idx_v, rows_v, t_v, gsem, ssem):
        wid = lax.axis_index("s") * info.num_cores + lax.axis_index("c")
        base = wid * rows_per_w
        pltpu.sync_copy(idx_hbm.at[pl.ds(base, rows_per_w)], idx_v)
        lanes = lax.iota(jnp.int32, 16)
        rvs = [m * 16 + lanes for m in range(8)]

        def fire(i, b):
            for j in range(_K):
                pltpu.async_copy(
                    table_hbm.at[idx_v.at[i * _K + j]],
                    rows_v.at[b].at[pl.ds(j * _ROW, _ROW)],
                    gsem.at[b],
                )

        def drain(b):
            for j in range(_K):
                pltpu.make_async_copy(
                    table_hbm.at[idx_v.at[0]],
                    rows_v.at[b].at[pl.ds(j * _ROW, _ROW)],
                    gsem.at[b],
                ).wait()

        def tile_wait(tb):
            for _ in range(ncg):
                pltpu.make_async_copy(
                    t_v.at[tb].at[0], out_hbm.at[pl.ds(0, D)], ssem.at[tb]
                ).wait()

        def tstore(i, b):
            for j in range(_K):
                g = base + i * _K + j
                h = g // ngrp_b
                bg = g % ngrp_b
                tb = j % _NT
                if j >= _NT:
                    tile_wait(tb)
                else:
                    @pl.when(i > 0)
                    def _():
                        tile_wait(tb)
                grp = rows_v.at[b].at[pl.ds(j * _ROW, _ROW)]

                def cbody(c, carry):
                    cvec = jnp.broadcast_to(c, (16,)).astype(jnp.int32)
                    for m in range(8):
                        vals = plsc.load_gather(grp, [rvs[m], cvec])
                        t_v[tb, c // 8, (c % 8) * 4 + m // 2,
                            pl.ds((m % 2) * 16, 16)] = vals
                    return carry

                lax.fori_loop(0, D, cbody, 0, unroll=2)
                for cg in range(ncg):
                    f0 = ((h * ncg + cg) * ngrp_b + bg) * D
                    pltpu.async_copy(
                        t_v.at[tb].at[cg], out_hbm.at[pl.ds(f0, D)],
                        ssem.at[tb],
                    )

        fire(0, 0)

        def pair(g2, carry):
            for sb in range(2):
                i = g2 * 2 + sb
                drain(sb)

                @pl.when(i + 1 < n_chunks)
                def _():
                    fire(i + 1, 1 - sb)

                tstore(i, sb)
            return carry

        lax.fori_loop(0, n_chunks // 2, pair, 0)
        drain(0)
        tstore(n_chunks - 1, 0)
        for tb in range(_NT):
            tile_wait(tb)

    return k


def kernel(input_ids, table):
    B, H = input_ids.shape
    V, D = table.shape
    nq = 128 // D
    tableP = _make_table_transpose(V, D)(table.T).reshape(-1, D)
    ids = input_ids.T.astype(jnp.int32)
    blk = nq * _RBL
    ids = nq * ((ids // blk) * _RBL + (ids % _RBL)) + (ids % blk) // _RBL
    idx = ids.reshape(-1, _ROW)
    flat = _make_gather(tableP.shape[0], D, B * H, H)(idx, tableP)
    out6 = flat.reshape(H, D // 8, B // 128, 8, 128)
    return out6.transpose(0, 1, 3, 2, 4).reshape(H, D, B).transpose(2, 0, 1)


# speedup vs baseline: 1.8721x; 1.8721x over previous
import functools

import jax
import jax.numpy as jnp
from jax import lax
from jax.experimental import pallas as pl
from jax.experimental.pallas import tpu as pltpu
from jax.experimental.pallas import tpu_sc as plsc

_ROW = 128
_K = 10
_NBUF = 2
_BB = 2048


@functools.lru_cache(maxsize=None)
def _make_gather(V, D, B):
    info = plsc.get_sparse_core_info()
    nw = info.num_cores * info.num_subcores
    assert B % (nw * _NBUF * _K * _ROW) == 0
    rows_per_w = B // (nw * _ROW)
    n_pairs = rows_per_w // (_K * _NBUF)
    chunk = _K * _ROW
    mesh = plsc.VectorSubcoreMesh(core_axis_name="c", subcore_axis_name="s")

    @functools.partial(
        pl.kernel,
        mesh=mesh,
        compiler_params=pltpu.CompilerParams(use_tc_tiling_on_sc=False),
        out_type=jax.ShapeDtypeStruct((B, D), jnp.float32),
        scratch_types=[
            pltpu.VMEM((rows_per_w, _ROW), jnp.int32),
            pltpu.VMEM((_NBUF, chunk, D), jnp.float32),
            pltpu.SemaphoreType.DMA,
            pltpu.SemaphoreType.DMA((_NBUF,)),
        ],
    )
    def k(idx_hbm, table_hbm, out_hbm, idx_v, rows_v, gsem, ssem):
        wid = lax.axis_index("s") * info.num_cores + lax.axis_index("c")
        base = wid * rows_per_w
        pltpu.sync_copy(idx_hbm.at[pl.ds(base, rows_per_w)], idx_v)

        def store_desc(b, flat0):
            return pltpu.make_async_copy(
                rows_v.at[b], out_hbm.at[pl.ds(flat0, chunk)], ssem.at[b]
            )

        def pair_body(g, carry):
            for b in range(_NBUF):
                i = g * _NBUF + b
                flat0 = (base + i * _K) * _ROW

                @pl.when(g > 0)
                def _():
                    store_desc(b, flat0).wait()

                copies = [
                    pltpu.async_copy(
                        table_hbm.at[idx_v.at[i * _K + j]],
                        rows_v.at[b].at[pl.ds(j * _ROW, _ROW)],
                        gsem,
                    )
                    for j in range(_K)
                ]
                for c in copies:
                    c.wait()
                store_desc(b, flat0).start()
            return carry

        lax.fori_loop(0, n_pairs, pair_body, 0)
        for b in range(_NBUF):
            store_desc(b, base * _ROW).wait()

    return k


@functools.lru_cache(maxsize=None)
def _make_retile(B, H, D):
    nq = 128 // D
    rb = B * D // 128

    def body(x_ref, o_ref):
        xT = x_ref[0].T
        o_ref[0] = jnp.concatenate(
            [xT[D * q:D * (q + 1)] for q in range(nq)], axis=1
        )

    return pl.pallas_call(
        body,
        grid=(H,),
        in_specs=[pl.BlockSpec((1, rb, 128), lambda h: (h, 0, 0))],
        out_specs=pl.BlockSpec((1, D, B), lambda h: (h, 0, 0)),
        out_shape=jax.ShapeDtypeStruct((H, D, B), jnp.float32),
    )


_RBL = 4096


@functools.lru_cache(maxsize=None)
def _make_table_transpose(V, D):
    nq = 128 // D
    nb = -(-V // (nq * _RBL))

    def body(x_ref, o_ref):
        x = x_ref[...]
        o_ref[...] = jnp.concatenate(
            [x[:, j * _RBL:(j + 1) * _RBL] for j in range(nq)], axis=0
        ).T

    return pl.pallas_call(
        body,
        grid=(nb,),
        in_specs=[pl.BlockSpec((D, nq * _RBL), lambda b: (0, b))],
        out_specs=pl.BlockSpec((_RBL, 128), lambda b: (b, 0)),
        out_shape=jax.ShapeDtypeStruct((nb * _RBL, 128), jnp.float32),
    )


def kernel(input_ids, table):
    B, H = input_ids.shape
    V, D = table.shape
    nq = 128 // D
    tableP = _make_table_transpose(V, D)(table.T).reshape(-1, D)
    ids = input_ids.T.astype(jnp.int32)
    blk = nq * _RBL
    ids = nq * ((ids // blk) * _RBL + (ids % _RBL)) + (ids % blk) // _RBL
    idx = (
        ids.reshape(H, nq, B // nq)
        .transpose(0, 2, 1)
        .reshape(-1, _ROW)
    )
    flat = _make_gather(tableP.shape[0], D, B * H)(idx, tableP)
    outT = _make_retile(B, H, D)(flat.reshape(H, -1, 128))
    return outT.transpose(2, 0, 1)


# speedup vs baseline: 1.9408x; 1.0367x over previous
import functools

import jax
import jax.numpy as jnp
from jax import lax
from jax.experimental import pallas as pl
from jax.experimental.pallas import tpu as pltpu
from jax.experimental.pallas import tpu_sc as plsc

_ROW = 128
_K = 10
_NBUF = 2
_BB = 2048


@functools.lru_cache(maxsize=None)
def _make_gather(V, D, B):
    info = plsc.get_sparse_core_info()
    nw = info.num_cores * info.num_subcores
    assert B % (nw * _NBUF * _K * _ROW) == 0
    rows_per_w = B // (nw * _ROW)
    n_pairs = rows_per_w // (_K * _NBUF)
    chunk = _K * _ROW
    mesh = plsc.VectorSubcoreMesh(core_axis_name="c", subcore_axis_name="s")

    @functools.partial(
        pl.kernel,
        mesh=mesh,
        compiler_params=pltpu.CompilerParams(use_tc_tiling_on_sc=False),
        out_type=jax.ShapeDtypeStruct((B, D), jnp.float32),
        scratch_types=[
            pltpu.VMEM((rows_per_w, _ROW), jnp.int32),
            pltpu.VMEM((_NBUF, chunk, D), jnp.float32),
            pltpu.SemaphoreType.DMA,
            pltpu.SemaphoreType.DMA((_NBUF,)),
        ],
    )
    def k(idx_hbm, table_hbm, out_hbm, idx_v, rows_v, gsem, ssem):
        wid = lax.axis_index("s") * info.num_cores + lax.axis_index("c")
        base = wid * rows_per_w
        pltpu.sync_copy(idx_hbm.at[pl.ds(base, rows_per_w)], idx_v)

        def store_desc(b, flat0):
            return pltpu.make_async_copy(
                rows_v.at[b], out_hbm.at[pl.ds(flat0, chunk)], ssem.at[b]
            )

        def pair_body(g, carry):
            for b in range(_NBUF):
                i = g * _NBUF + b
                flat0 = (base + i * _K) * _ROW

                @pl.when(g > 0)
                def _():
                    store_desc(b, flat0).wait()

                copies = [
                    pltpu.async_copy(
                        table_hbm.at[idx_v.at[i * _K + j]],
                        rows_v.at[b].at[pl.ds(j * _ROW, _ROW)],
                        gsem,
                    )
                    for j in range(_K)
                ]
                for c in copies:
                    c.wait()
                store_desc(b, flat0).start()
            return carry

        lax.fori_loop(0, n_pairs, pair_body, 0)
        for b in range(_NBUF):
            store_desc(b, base * _ROW).wait()

    return k


@functools.lru_cache(maxsize=None)
def _make_retile(B, H, D):
    nq = 128 // D
    rb = B * D // 128

    def body(x_ref, o_ref):
        xT = x_ref[0].T
        o_ref[0] = jnp.concatenate(
            [xT[D * q:D * (q + 1)] for q in range(nq)], axis=1
        )

    return pl.pallas_call(
        body,
        grid=(H,),
        in_specs=[pl.BlockSpec((1, rb, 128), lambda h: (h, 0, 0))],
        out_specs=pl.BlockSpec((1, D, B), lambda h: (h, 0, 0)),
        out_shape=jax.ShapeDtypeStruct((H, D, B), jnp.float32),
    )


_RBL = 8192


@functools.lru_cache(maxsize=None)
def _make_table_transpose(V, D):
    nq = 128 // D
    nb = -(-V // (nq * _RBL))

    def body(x_ref, o_ref):
        x = x_ref[...]
        o_ref[...] = jnp.concatenate(
            [x[:, j * _RBL:(j + 1) * _RBL] for j in range(nq)], axis=0
        ).T

    return pl.pallas_call(
        body,
        grid=(nb,),
        in_specs=[pl.BlockSpec((D, nq * _RBL), lambda b: (0, b))],
        out_specs=pl.BlockSpec((_RBL, 128), lambda b: (b, 0)),
        out_shape=jax.ShapeDtypeStruct((nb * _RBL, 128), jnp.float32),
    )


def kernel(input_ids, table):
    B, H = input_ids.shape
    V, D = table.shape
    nq = 128 // D
    tableP = _make_table_transpose(V, D)(table.T).reshape(-1, D)
    ids = input_ids.T.astype(jnp.int32)
    blk = nq * _RBL
    ids = nq * ((ids // blk) * _RBL + (ids % _RBL)) + (ids % blk) // _RBL
    idx = (
        ids.reshape(H, nq, B // nq)
        .transpose(0, 2, 1)
        .reshape(-1, _ROW)
    )
    flat = _make_gather(tableP.shape[0], D, B * H)(idx, tableP)
    outT = _make_retile(B, H, D)(flat.reshape(H, -1, 128))
    return outT.transpose(2, 0, 1)


# speedup vs baseline: 1.9503x; 1.0049x over previous
import functools

import jax
import jax.numpy as jnp
from jax import lax
from jax.experimental import pallas as pl
from jax.experimental.pallas import tpu as pltpu
from jax.experimental.pallas import tpu_sc as plsc

_ROW = 128
_K = 10
_NBUF = 2
_BB = 2048


@functools.lru_cache(maxsize=None)
def _make_gather(V, D, B):
    info = plsc.get_sparse_core_info()
    nw = info.num_cores * info.num_subcores
    assert B % (nw * _NBUF * _K * _ROW) == 0
    rows_per_w = B // (nw * _ROW)
    n_pairs = rows_per_w // (_K * _NBUF)
    chunk = _K * _ROW
    mesh = plsc.VectorSubcoreMesh(core_axis_name="c", subcore_axis_name="s")

    @functools.partial(
        pl.kernel,
        mesh=mesh,
        compiler_params=pltpu.CompilerParams(use_tc_tiling_on_sc=False),
        out_type=jax.ShapeDtypeStruct((B, D), jnp.float32),
        scratch_types=[
            pltpu.VMEM((rows_per_w, _ROW), jnp.int32),
            pltpu.VMEM((_NBUF, chunk, D), jnp.float32),
            pltpu.SemaphoreType.DMA,
            pltpu.SemaphoreType.DMA((_NBUF,)),
        ],
    )
    def k(idx_hbm, table_hbm, out_hbm, idx_v, rows_v, gsem, ssem):
        wid = lax.axis_index("s") * info.num_cores + lax.axis_index("c")
        base = wid * rows_per_w
        pltpu.sync_copy(idx_hbm.at[pl.ds(base, rows_per_w)], idx_v)

        def store_desc(b, flat0):
            return pltpu.make_async_copy(
                rows_v.at[b], out_hbm.at[pl.ds(flat0, chunk)], ssem.at[b]
            )

        def pair_body(g, carry):
            for b in range(_NBUF):
                i = g * _NBUF + b
                flat0 = (base + i * _K) * _ROW

                @pl.when(g > 0)
                def _():
                    store_desc(b, flat0).wait()

                copies = [
                    pltpu.async_copy(
                        table_hbm.at[idx_v.at[i * _K + j]],
                        rows_v.at[b].at[pl.ds(j * _ROW, _ROW)],
                        gsem,
                    )
                    for j in range(_K)
                ]
                for c in copies:
                    c.wait()
                store_desc(b, flat0).start()
            return carry

        lax.fori_loop(0, n_pairs, pair_body, 0)
        for b in range(_NBUF):
            store_desc(b, base * _ROW).wait()

    return k


@functools.lru_cache(maxsize=None)
def _make_retile(B, H, D):
    nq = 128 // D
    rb = B * D // 128

    def body(x_ref, o_ref):
        xT = x_ref[0].T
        o_ref[0] = jnp.concatenate(
            [xT[D * q:D * (q + 1)] for q in range(nq)], axis=1
        )

    return pl.pallas_call(
        body,
        grid=(H,),
        in_specs=[pl.BlockSpec((1, rb, 128), lambda h: (h, 0, 0))],
        out_specs=pl.BlockSpec((1, D, B), lambda h: (h, 0, 0)),
        out_shape=jax.ShapeDtypeStruct((H, D, B), jnp.float32),
    )


_RBL = 16384


@functools.lru_cache(maxsize=None)
def _make_table_transpose(V, D):
    nq = 128 // D
    nb = -(-V // (nq * _RBL))

    def body(x_ref, o_ref):
        x = x_ref[...]
        o_ref[...] = jnp.concatenate(
            [x[:, j * _RBL:(j + 1) * _RBL] for j in range(nq)], axis=0
        ).T

    return pl.pallas_call(
        body,
        grid=(nb,),
        in_specs=[pl.BlockSpec((D, nq * _RBL), lambda b: (0, b))],
        out_specs=pl.BlockSpec((_RBL, 128), lambda b: (b, 0)),
        out_shape=jax.ShapeDtypeStruct((nb * _RBL, 128), jnp.float32),
    )


def kernel(input_ids, table):
    B, H = input_ids.shape
    V, D = table.shape
    nq = 128 // D
    tableP = _make_table_transpose(V, D)(table.T).reshape(-1, D)
    ids = input_ids.T.astype(jnp.int32)
    blk = nq * _RBL
    ids = nq * ((ids // blk) * _RBL + (ids % _RBL)) + (ids % blk) // _RBL
    idx = (
        ids.reshape(H, nq, B // nq)
        .transpose(0, 2, 1)
        .reshape(-1, _ROW)
    )
    flat = _make_gather(tableP.shape[0], D, B * H)(idx, tableP)
    outT = _make_retile(B, H, D)(flat.reshape(H, -1, 128))
    return outT.transpose(2, 0, 1)
